# baseline (speedup 1.0000x reference)
import functools

import jax
import jax.numpy as jnp
from jax import lax
from jax.experimental import pallas as pl
from jax.experimental.pallas import tpu as pltpu
from jax.experimental.pallas import tpu_sc as plsc


def _make_gather(V, C, B):
    info = plsc.get_sparse_core_info()
    NC, NS = info.num_cores, info.num_subcores
    NW = NC * NS
    assert B % (8 * NW) == 0
    b_per_w = B // NW
    mesh = plsc.VectorSubcoreMesh(core_axis_name="c", subcore_axis_name="s")

    @functools.partial(
        pl.kernel,
        mesh=mesh,
        out_type=jax.ShapeDtypeStruct((B, C), jnp.float32),
        scratch_types=[
            pltpu.VMEM((b_per_w,), jnp.int32),
            pltpu.VMEM((b_per_w, C), jnp.float32),
            pltpu.SemaphoreType.DMA,
        ],
        compiler_params=pltpu.CompilerParams(use_tc_tiling_on_sc=False),
    )
    def gather_kernel(table_hbm, idx_hbm, out_hbm, idx_v, rows_v, sem):
        wid = lax.axis_index("s") * NC + lax.axis_index("c")
        base = wid * b_per_w
        pltpu.sync_copy(idx_hbm.at[pl.ds(base, b_per_w)], idx_v)
        pltpu.async_copy(table_hbm.at[idx_v], rows_v, sem).wait()
        pltpu.sync_copy(rows_v, out_hbm.at[pl.ds(base, b_per_w)])

    return gather_kernel


def kernel(genre, table):
    B, = genre.shape
    V, C = table.shape
    out = _make_gather(V, C, B)(table, genre)
    return out[:, :, None, None]


# speedup vs baseline: 1.9768x; 1.9768x over previous
import functools

import jax
import jax.numpy as jnp
from jax import lax
from jax.experimental import pallas as pl
from jax.experimental.pallas import tpu as pltpu
from jax.experimental.pallas import tpu_sc as plsc


def _make_lookup(V, C, B):
    info = plsc.get_sparse_core_info()
    NC, NS, L = info.num_cores, info.num_subcores, info.num_lanes
    NW = NC * NS
    assert C == NW
    CHUNK = 8192
    n_chunks = B // CHUNK
    mesh = plsc.VectorSubcoreMesh(core_axis_name="c", subcore_axis_name="s")

    @functools.partial(
        pl.kernel,
        mesh=mesh,
        out_type=jax.ShapeDtypeStruct((C, B), jnp.float32),
        scratch_types=[
            pltpu.VMEM((V,), jnp.float32),
            pltpu.VMEM((CHUNK,), jnp.int32),
            pltpu.VMEM((CHUNK,), jnp.float32),
        ],
        compiler_params=pltpu.CompilerParams(
            use_tc_tiling_on_sc=True, needs_layout_passes=False
        ),
    )
    def lookup_kernel(tableT_hbm, idx_hbm, outT_hbm, row_v, idx_v, val_v):
        ch = lax.axis_index("s") * NC + lax.axis_index("c")
        pltpu.sync_copy(tableT_hbm.at[ch], row_v)

        def chunk_body(k, _):
            base = k * CHUNK
            pltpu.sync_copy(idx_hbm.at[pl.ds(base, CHUNK)], idx_v)

            def gather_body(j, _):
                idxs = idx_v[pl.ds(j * L, L)]
                val_v[pl.ds(j * L, L)] = plsc.load_gather(row_v, [idxs])
                return ()

            lax.fori_loop(0, CHUNK // L, gather_body, (), unroll=8)
            pltpu.sync_copy(val_v, outT_hbm.at[ch, pl.ds(base, CHUNK)])
            return ()

        lax.fori_loop(0, n_chunks, chunk_body, ())

    return lookup_kernel


def kernel(genre, table):
    B, = genre.shape
    V, C = table.shape
    outT = _make_lookup(V, C, B)(table.T, genre)
    return outT.T[:, :, None, None]


# speedup vs baseline: 2.0660x; 1.0452x over previous
import functools

import jax
import jax.numpy as jnp
from jax import lax
from jax.experimental import pallas as pl
from jax.experimental.pallas import tpu as pltpu
from jax.experimental.pallas import tpu_sc as plsc


def _make_lookup(V, C, B):
    info = plsc.get_sparse_core_info()
    NC, NS, L = info.num_cores, info.num_subcores, info.num_lanes
    NW = NC * NS
    assert C == NW
    CHUNK = 4096
    n_chunks = B // CHUNK
    mesh = plsc.VectorSubcoreMesh(core_axis_name="c", subcore_axis_name="s")

    @functools.partial(
        pl.kernel,
        mesh=mesh,
        out_type=jax.ShapeDtypeStruct((C, B), jnp.float32),
        scratch_types=[
            pltpu.VMEM((V,), jnp.float32),
            pltpu.VMEM((B,), jnp.int32),
            pltpu.VMEM((CHUNK,), jnp.float32),
            pltpu.VMEM((CHUNK,), jnp.float32),
            pltpu.SemaphoreType.DMA,
            pltpu.SemaphoreType.DMA,
            pltpu.SemaphoreType.DMA,
        ],
        compiler_params=pltpu.CompilerParams(
            use_tc_tiling_on_sc=True, needs_layout_passes=False
        ),
    )
    def lookup_kernel(tableT_hbm, idx_hbm, outT_hbm, row_v, idx_v, val_a,
                      val_b, sem_row, sem_idx, sem_out):
        ch = lax.axis_index("s") * NC + lax.axis_index("c")
        row_cp = pltpu.async_copy(tableT_hbm.at[ch], row_v, sem_row)
        idx_cp = pltpu.async_copy(idx_hbm, idx_v, sem_idx)
        idx_cp.wait()
        row_cp.wait()

        bufs = (val_a, val_b)
        for k in range(n_chunks):
            buf = bufs[k % 2]
            if k >= 2:
                pltpu.make_async_copy(
                    buf, outT_hbm.at[ch, pl.ds((k - 2) * CHUNK, CHUNK)], sem_out
                ).wait()

            def gather_body(j, _, k=k, buf=buf):
                idxs = idx_v[pl.ds(k * CHUNK + j * L, L)]
                buf[pl.ds(j * L, L)] = plsc.load_gather(row_v, [idxs])
                return ()

            lax.fori_loop(0, CHUNK // L, gather_body, (), unroll=16)
            pltpu.async_copy(
                buf, outT_hbm.at[ch, pl.ds(k * CHUNK, CHUNK)], sem_out
            )
        for k in range(n_chunks - 2, n_chunks):
            pltpu.make_async_copy(
                bufs[k % 2], outT_hbm.at[ch, pl.ds(k * CHUNK, CHUNK)], sem_out
            ).wait()

    return lookup_kernel


def kernel(genre, table):
    B, = genre.shape
    V, C = table.shape
    outT = _make_lookup(V, C, B)(table.T, genre)
    return outT.T[:, :, None, None]


# speedup vs baseline: 2.4620x; 1.1917x over previous
import functools

import jax
import jax.numpy as jnp
from jax import lax
from jax.experimental import pallas as pl
from jax.experimental.pallas import tpu as pltpu
from jax.experimental.pallas import tpu_sc as plsc


def _make_lookup(V, C, B):
    info = plsc.get_sparse_core_info()
    NC, NS, L = info.num_cores, info.num_subcores, info.num_lanes
    NW = NC * NS
    assert C == NW
    CHUNK = 4096
    n_chunks = B // CHUNK
    U = 8
    mesh = plsc.VectorSubcoreMesh(core_axis_name="c", subcore_axis_name="s")

    @functools.partial(
        pl.kernel,
        mesh=mesh,
        out_type=jax.ShapeDtypeStruct((C, B), jnp.float32),
        scratch_types=[
            pltpu.VMEM((V,), jnp.float32),
            pltpu.VMEM((B,), jnp.int32),
            pltpu.VMEM((CHUNK,), jnp.float32),
            pltpu.VMEM((CHUNK,), jnp.float32),
            pltpu.SemaphoreType.DMA,
            pltpu.SemaphoreType.DMA,
            pltpu.SemaphoreType.DMA,
        ],
        compiler_params=pltpu.CompilerParams(
            use_tc_tiling_on_sc=True, needs_layout_passes=False
        ),
    )
    def lookup_kernel(tableT_hbm, idx_hbm, outT_hbm, row_v, idx_v, val_a,
                      val_b, sem_row, sem_idx, sem_out):
        ch = lax.axis_index("s") * NC + lax.axis_index("c")
        row_cp = pltpu.async_copy(tableT_hbm.at[ch], row_v, sem_row)
        idx_cp = pltpu.async_copy(idx_hbm, idx_v, sem_idx)
        idx_cp.wait()
        row_cp.wait()

        bufs = (val_a, val_b)
        for k in range(n_chunks):
            buf = bufs[k % 2]
            if k >= 2:
                pltpu.make_async_copy(
                    buf, outT_hbm.at[ch, pl.ds((k - 2) * CHUNK, CHUNK)], sem_out
                ).wait()

            def gather_body(j, _, k=k, buf=buf):
                base = k * CHUNK + j * (L * U)
                idx_vecs = [idx_v[pl.ds(base + u * L, L)] for u in range(U)]
                vals = [plsc.load_gather(row_v, [iv]) for iv in idx_vecs]
                for u in range(U):
                    buf[pl.ds(j * (L * U) + u * L, L)] = vals[u]
                return ()

            lax.fori_loop(0, CHUNK // (L * U), gather_body, (), unroll=2)
            pltpu.async_copy(
                buf, outT_hbm.at[ch, pl.ds(k * CHUNK, CHUNK)], sem_out
            )
        for k in range(n_chunks - 2, n_chunks):
            pltpu.make_async_copy(
                bufs[k % 2], outT_hbm.at[ch, pl.ds(k * CHUNK, CHUNK)], sem_out
            ).wait()

    return lookup_kernel


def kernel(genre, table):
    B, = genre.shape
    V, C = table.shape
    outT = _make_lookup(V, C, B)(table.T, genre)
    return outT.T[:, :, None, None]
